# baseline (speedup 1.0000x reference)
import functools

import jax
import jax.numpy as jnp
from jax import lax
from jax.experimental import pallas as pl
from jax.experimental.pallas import tpu as pltpu
from jax.experimental.pallas import tpu_sc as plsc

NUM_CORES = 2
NUM_SUBCORES = 16
NUM_WORKERS = NUM_CORES * NUM_SUBCORES
ROWS_PER_STEP = 32


def _emb_call(n_seq, seq_len, d):
    mesh = plsc.VectorSubcoreMesh(core_axis_name="c", subcore_axis_name="s")
    seq_per_worker = n_seq // NUM_WORKERS
    n_steps = seq_per_worker // ROWS_PER_STEP

    @functools.partial(
        pl.kernel,
        out_type=jax.ShapeDtypeStruct((n_seq, seq_len, d), jnp.float32),
        mesh=mesh,
        scratch_types=[
            pltpu.VMEM((seq_per_worker, seq_len), jnp.int32),
            pltpu.VMEM((2, ROWS_PER_STEP, seq_len, d), jnp.float32),
            pltpu.SemaphoreType.DMA,
            pltpu.SemaphoreType.DMA,
            pltpu.SemaphoreType.DMA,
            pltpu.SemaphoreType.DMA,
        ],
        compiler_params=pltpu.CompilerParams(use_tc_tiling_on_sc=False),
    )
    def emb(x_hbm, w_hbm, out_hbm, idx_v, rows_v, g0, g1, w0, w1):
        wid = lax.axis_index("s") * NUM_CORES + lax.axis_index("c")
        base = wid * seq_per_worker
        pltpu.sync_copy(x_hbm.at[pl.ds(base, seq_per_worker)], idx_v)
        gsems = (g0, g1)
        wsems = (w0, w1)

        def fire(s, buf):
            for m in range(ROWS_PER_STEP):
                pltpu.async_copy(
                    w_hbm.at[idx_v.at[s * ROWS_PER_STEP + m]],
                    rows_v.at[buf, m],
                    gsems[buf])

        def drain_gathers(buf):
            pltpu.make_async_copy(
                out_hbm.at[pl.ds(0, ROWS_PER_STEP)], rows_v.at[buf],
                gsems[buf]).wait()

        def wait_writeback(s, buf):
            pltpu.make_async_copy(
                rows_v.at[buf],
                out_hbm.at[pl.ds(base + s * ROWS_PER_STEP, ROWS_PER_STEP)],
                wsems[buf]).wait()

        def do_step(s, buf):
            drain_gathers(buf)
            pltpu.async_copy(
                rows_v.at[buf],
                out_hbm.at[pl.ds(base + s * ROWS_PER_STEP, ROWS_PER_STEP)],
                wsems[buf])
            nxt = buf ^ 1

            @pl.when(s > 0)
            def _():
                wait_writeback(s - 1, nxt)

            @pl.when(s + 1 < n_steps)
            def _():
                fire(s + 1, nxt)

        fire(0, 0)

        def body(i, carry):
            do_step(2 * i, 0)
            do_step(2 * i + 1, 1)
            return carry

        lax.fori_loop(0, n_steps // 2, body, 0)
        wait_writeback(n_steps - 1, (n_steps - 1) % 2)

    return emb


def kernel(x, weight):
    n_seq, seq_len = x.shape
    d = weight.shape[1]
    assert n_seq % (NUM_WORKERS * ROWS_PER_STEP) == 0
    return _emb_call(n_seq, seq_len, d)(x.astype(jnp.int32), weight)
